# Initial kernel scaffold; baseline (speedup 1.0000x reference)
import jax
import jax.numpy as jnp
from jax.experimental import pallas as pl


def kernel(s, v, radial_embeddings_1, radial_embeddings_2, f_cut_1, f_cut_2, unit_vectors_1, unit_vectors_2, edge_index, W1, b1, W2, b2, Wr, br):
    raise NotImplementedError("write your pallas kernel here")


# speedup vs baseline: 1.2415x; 1.2415x over previous
import functools

import jax
import jax.numpy as jnp
from jax import lax
from jax.experimental import pallas as pl
from jax.experimental.pallas import tpu as pltpu
from jax.experimental.pallas import tpu_sc as plsc

N = 10000
E = 320000
F = 128
R = 16
R4F = 512
GW_V = 3 * F

NC, NS, L = 2, 16, 16
NW = NC * NS
EPW = E // NW
CH = 40
NCHUNK = EPW // CH
NROW = 640
N_PAD = NROW * NS
ZR = 40
NROW_LAST = N - NROW * (NS - 1)

BN = 2000
BE = 4000


def _node_tc_body(s_ref, v_ref, w1t_ref, b1_ref, w2t_ref, b2_ref, gs_ref, gv_ref):
    h = jnp.dot(s_ref[...], w1t_ref[...], preferred_element_type=jnp.float32)
    h = h + b1_ref[...]
    h = h * jax.nn.sigmoid(h)
    phi = jnp.dot(h, w2t_ref[...], preferred_element_type=jnp.float32) + b2_ref[...]
    gs_ref[...] = phi[:, :F]
    gv_ref[...] = jnp.concatenate(
        [phi[:, F:2 * F] * v_ref[...], phi[:, 2 * F:3 * F], phi[:, 3 * F:4 * F]],
        axis=1)


def _edge_tc_body(r1_ref, r2_ref, fc1_ref, fc2_ref, u1_ref, u2_ref, wrt_ref,
                  br_ref, ws_ref, wv_ref):
    fc1 = fc1_ref[...]
    fc2 = fc2_ref[...]
    a = fc1 * r1_ref[...] + fc2 * r2_ref[...]
    w = jnp.dot(a, wrt_ref[...], preferred_element_type=jnp.float32)
    w = w + (fc1 + fc2) * br_ref[...]
    ws_ref[...] = w[:, :F]
    wv_ref[...] = jnp.concatenate(
        [w[:, F:2 * F], w[:, 2 * F:3 * F] * u1_ref[...],
         w[:, 3 * F:4 * F] * u2_ref[...]],
        axis=1)


def _combine_tc_body(s_ref, v_ref, dsp_ref, dvp_ref, os_ref, ov_ref):
    os_ref[...] = s_ref[...] + dsp_ref[0] + dsp_ref[1]
    ov_ref[...] = v_ref[...] + dvp_ref[0] + dvp_ref[1]


_sc_mesh = plsc.VectorSubcoreMesh(core_axis_name="c", subcore_axis_name="s")


@functools.partial(
    pl.kernel,
    out_type=(jax.ShapeDtypeStruct((NC, N, F), jnp.float32),
              jax.ShapeDtypeStruct((NC, N, F), jnp.float32)),
    mesh=_sc_mesh,
    scratch_types=[
        pltpu.VMEM((CH,), jnp.int32),
        pltpu.VMEM((CH,), jnp.int32),
        pltpu.VMEM((CH, F), jnp.float32),
        pltpu.VMEM((CH, GW_V), jnp.float32),
        pltpu.VMEM((CH, GW_V), jnp.float32),
        pltpu.VMEM((CH, F), jnp.float32),
        pltpu.VMEM((ZR, F), jnp.float32),
        pltpu.VMEM_SHARED((N_PAD, F), jnp.float32),
        pltpu.SemaphoreType.DMA,
    ],
)
def _sc_scatter_both(gs_hbm, ws_hbm, gv_hbm, wv_hbm, idxj_hbm, idxi_hbm,
                     ds_hbm, dv_hbm,
                     idxj_v, idxi_v, gs_v, gv_v, wv_v, x_v, z_v, acc, sem):
    cid = lax.axis_index("c")
    sid = lax.axis_index("s")
    nbase = pl.multiple_of(sid * NROW, 8)
    wid = sid * NC + cid
    wbase = wid * EPW

    zero = jnp.zeros((L,), jnp.float32)

    def zrow(rr, carry):
        for k in range(F // L):
            z_v[rr, pl.ds(k * L, L)] = zero
        return carry

    lax.fori_loop(0, ZR, zrow, 0)

    def zero_acc():
        for q in range(NROW // ZR):
            pltpu.sync_copy(z_v, acc.at[pl.ds(nbase + q * ZR, ZR)])

    def run_phase(g_hbm, w_hbm, g_v, w_v, msg_v, nblk):
        def chunk(t, carry):
            e0 = pl.multiple_of(wbase + t * CH, 8)
            pltpu.sync_copy(idxj_hbm.at[pl.ds(e0, CH)], idxj_v)
            pltpu.sync_copy(idxi_hbm.at[pl.ds(e0, CH)], idxi_v)
            pltpu.async_copy(g_hbm.at[idxj_v], g_v, sem).wait()
            pltpu.sync_copy(w_hbm.at[pl.ds(e0, CH)], w_v)

            def edge(c, icarry):
                for k in range(F // L):
                    a16 = g_v[c, pl.ds(k * L, L)] * w_v[c, pl.ds(k * L, L)]
                    for blk in range(1, nblk):
                        off = blk * F + k * L
                        a16 = a16 + g_v[c, pl.ds(off, L)] * w_v[c, pl.ds(off, L)]
                    msg_v[c, pl.ds(k * L, L)] = a16
                return icarry

            lax.fori_loop(0, CH, edge, 0)
            pltpu.sync_copy(msg_v, acc.at[idxi_v], add=True)
            return carry

        lax.fori_loop(0, NCHUNK, chunk, 0)

    def copy_out(out_hbm):
        @pl.when(sid != NS - 1)
        def _copy_full():
            pltpu.sync_copy(acc.at[pl.ds(nbase, NROW)],
                            out_hbm.at[cid, pl.ds(nbase, NROW)])

        @pl.when(sid == NS - 1)
        def _copy_tail():
            pltpu.sync_copy(acc.at[pl.ds(nbase, NROW_LAST)],
                            out_hbm.at[cid, pl.ds(nbase, NROW_LAST)])

    zero_acc()
    plsc.subcore_barrier()
    run_phase(gs_hbm, ws_hbm, gs_v, x_v, x_v, 1)
    plsc.subcore_barrier()
    copy_out(ds_hbm)
    zero_acc()
    plsc.subcore_barrier()
    run_phase(gv_hbm, wv_hbm, gv_v, wv_v, gs_v, GW_V // F)
    plsc.subcore_barrier()
    copy_out(dv_hbm)


def kernel(s, v, radial_embeddings_1, radial_embeddings_2, f_cut_1, f_cut_2,
           unit_vectors_1, unit_vectors_2, edge_index, W1, b1, W2, b2, Wr, br):
    idx_i = edge_index[0].astype(jnp.int32)
    idx_j = edge_index[1].astype(jnp.int32)
    fc1 = f_cut_1.reshape(E, 1)
    fc2 = f_cut_2.reshape(E, 1)
    u1 = unit_vectors_1.reshape(E, 1)
    u2 = unit_vectors_2.reshape(E, 1)

    gs, gv = pl.pallas_call(
        _node_tc_body,
        grid=(N // BN,),
        in_specs=[
            pl.BlockSpec((BN, F), lambda i: (i, 0)),
            pl.BlockSpec((BN, F), lambda i: (i, 0)),
            pl.BlockSpec((F, F), lambda i: (0, 0)),
            pl.BlockSpec((1, F), lambda i: (0, 0)),
            pl.BlockSpec((F, R4F), lambda i: (0, 0)),
            pl.BlockSpec((1, R4F), lambda i: (0, 0)),
        ],
        out_specs=[
            pl.BlockSpec((BN, F), lambda i: (i, 0)),
            pl.BlockSpec((BN, GW_V), lambda i: (i, 0)),
        ],
        out_shape=[
            jax.ShapeDtypeStruct((N, F), jnp.float32),
            jax.ShapeDtypeStruct((N, GW_V), jnp.float32),
        ],
    )(s, v, W1.T, b1.reshape(1, F), W2.T, b2.reshape(1, R4F))

    ws, wv = pl.pallas_call(
        _edge_tc_body,
        grid=(E // BE,),
        in_specs=[
            pl.BlockSpec((BE, R), lambda i: (i, 0)),
            pl.BlockSpec((BE, R), lambda i: (i, 0)),
            pl.BlockSpec((BE, 1), lambda i: (i, 0)),
            pl.BlockSpec((BE, 1), lambda i: (i, 0)),
            pl.BlockSpec((BE, 1), lambda i: (i, 0)),
            pl.BlockSpec((BE, 1), lambda i: (i, 0)),
            pl.BlockSpec((R, R4F), lambda i: (0, 0)),
            pl.BlockSpec((1, R4F), lambda i: (0, 0)),
        ],
        out_specs=[
            pl.BlockSpec((BE, F), lambda i: (i, 0)),
            pl.BlockSpec((BE, GW_V), lambda i: (i, 0)),
        ],
        out_shape=[
            jax.ShapeDtypeStruct((E, F), jnp.float32),
            jax.ShapeDtypeStruct((E, GW_V), jnp.float32),
        ],
    )(radial_embeddings_1, radial_embeddings_2, fc1, fc2, u1, u2,
      Wr.T, br.reshape(1, R4F))

    dsp, dvp = _sc_scatter_both(gs, ws, gv, wv, idx_j, idx_i)

    out_s, out_v = pl.pallas_call(
        _combine_tc_body,
        grid=(N // BN,),
        in_specs=[
            pl.BlockSpec((BN, F), lambda i: (i, 0)),
            pl.BlockSpec((BN, F), lambda i: (i, 0)),
            pl.BlockSpec((NC, BN, F), lambda i: (0, i, 0)),
            pl.BlockSpec((NC, BN, F), lambda i: (0, i, 0)),
        ],
        out_specs=[
            pl.BlockSpec((BN, F), lambda i: (i, 0)),
            pl.BlockSpec((BN, F), lambda i: (i, 0)),
        ],
        out_shape=[
            jax.ShapeDtypeStruct((N, F), jnp.float32),
            jax.ShapeDtypeStruct((N, F), jnp.float32),
        ],
    )(s, v, dsp, dvp)

    return out_s, out_v


# speedup vs baseline: 2.0321x; 1.6369x over previous
import functools

import jax
import jax.numpy as jnp
from jax import lax
from jax.experimental import pallas as pl
from jax.experimental.pallas import tpu as pltpu
from jax.experimental.pallas import tpu_sc as plsc

N = 10000
E = 320000
F = 128
R = 16
R4F = 512
NT = 4

NC, NS, L = 2, 16, 16
CH = 40
NROWS_ALL = E // CH
RPT = NROWS_ALL // NS
NBK = 20
NBLK = RPT // NBK
NROW = 640
N_PAD = NROW * NS
ZR = 40
NROW_LAST = N - NROW * (NS - 1)

BN = 2000
BE = 4000


def _node_tc_body(s_ref, v_ref, w1t_ref, b1_ref, w2t_ref, b2_ref, g_ref):
    h = jnp.dot(s_ref[...], w1t_ref[...], preferred_element_type=jnp.float32)
    h = h + b1_ref[...]
    h = h * jax.nn.sigmoid(h)
    phi = jnp.dot(h, w2t_ref[...], preferred_element_type=jnp.float32) + b2_ref[...]
    g_ref[0] = phi[:, :F]
    g_ref[1] = phi[:, F:2 * F] * v_ref[...]
    g_ref[2] = phi[:, 2 * F:3 * F]
    g_ref[3] = phi[:, 3 * F:4 * F]


def _edge_tc_body(r1_ref, r2_ref, fc1_ref, fc2_ref, u1_ref, u2_ref, wrt_ref,
                  br_ref, w_ref):
    fc1 = fc1_ref[...]
    fc2 = fc2_ref[...]
    a = fc1 * r1_ref[...] + fc2 * r2_ref[...]
    w = jnp.dot(a, wrt_ref[...], preferred_element_type=jnp.float32)
    w = w + (fc1 + fc2) * br_ref[...]
    w_ref[0] = w[:, :F]
    w_ref[1] = w[:, F:2 * F]
    w_ref[2] = w[:, 2 * F:3 * F] * u1_ref[...]
    w_ref[3] = w[:, 3 * F:4 * F] * u2_ref[...]


def _combine_tc_body(s_ref, v_ref, p_ref, os_ref, ov_ref):
    os_ref[...] = s_ref[...] + p_ref[0]
    ov_ref[...] = v_ref[...] + (p_ref[1] + p_ref[2] + p_ref[3])


_sc_mesh = plsc.VectorSubcoreMesh(core_axis_name="c", subcore_axis_name="s")


@functools.partial(
    pl.kernel,
    out_type=jax.ShapeDtypeStruct((NT, N, F), jnp.float32),
    mesh=_sc_mesh,
    scratch_types=[
        pltpu.VMEM((NBK, CH), jnp.int32),
        pltpu.VMEM((NBK, CH), jnp.int32),
        pltpu.VMEM((CH, F), jnp.float32),
        pltpu.VMEM((CH, F), jnp.float32),
        pltpu.VMEM((CH, F), jnp.float32),
        pltpu.VMEM((CH, F), jnp.float32),
        pltpu.VMEM((CH, F), jnp.float32),
        pltpu.VMEM((CH, F), jnp.float32),
        pltpu.VMEM((ZR, F), jnp.float32),
        pltpu.VMEM_SHARED((N_PAD, F), jnp.float32),
        pltpu.SemaphoreType.DMA,
        pltpu.SemaphoreType.DMA,
        pltpu.SemaphoreType.DMA,
        pltpu.SemaphoreType.DMA,
    ],
)
def _sc_scatter4(g_hbm, w_hbm, idxj_hbm, idxi_hbm, out_hbm,
                 idxj_blk, idxi_blk, g_a, g_b, w_a, w_b, m_a, m_b, z_v, acc,
                 sem_ga, sem_gb, sem_wa, sem_wb):
    cid = lax.axis_index("c")
    sid = lax.axis_index("s")
    nbase = pl.multiple_of(sid * NROW, 8)

    zero = jnp.zeros((L,), jnp.float32)

    def zrow(rr, carry):
        for k in range(F // L):
            z_v[rr, pl.ds(k * L, L)] = zero
        return carry

    lax.fori_loop(0, ZR, zrow, 0)

    def zero_acc():
        for q in range(NROW // ZR):
            pltpu.sync_copy(z_v, acc.at[pl.ds(nbase + q * ZR, ZR)])

    def compute(g_v, w_v, m_v):
        def edge(c, icarry):
            for k in range(F // L):
                m_v[c, pl.ds(k * L, L)] = (
                    g_v[c, pl.ds(k * L, L)] * w_v[c, pl.ds(k * L, L)])
            return icarry

        lax.fori_loop(0, CH, edge, 0)

    def run_task(tid):
        rbase = sid * RPT

        def issue(row, ebase, g_v, w_v, sem_g, sem_w):
            e0 = pl.multiple_of(ebase, 8)
            gd = pltpu.async_copy(g_hbm.at[idxj_blk.at[row]], g_v, sem_g)
            wd = pltpu.async_copy(w_hbm.at[pl.ds(e0, CH)], w_v, sem_w)
            return gd, wd

        def wait(g_v, w_v, sem_g, sem_w):
            pltpu.make_async_copy(g_hbm.at[pl.ds(0, CH)], g_v, sem_g).wait()
            pltpu.make_async_copy(w_hbm.at[pl.ds(0, CH)], w_v, sem_w).wait()

        def block(blk, carry):
            q0 = rbase + blk * NBK
            pltpu.sync_copy(idxj_hbm.at[tid, sid, blk], idxj_blk)
            pltpu.sync_copy(idxi_hbm.at[sid, blk], idxi_blk)
            ebase0 = tid * E + q0 * CH
            issue(0, ebase0, g_a, w_a, sem_ga, sem_wa)

            def pair(k, icarry):
                r0 = 2 * k
                eb0 = ebase0 + r0 * CH
                issue(r0 + 1, eb0 + CH, g_b, w_b, sem_gb, sem_wb)
                wait(g_a, w_a, sem_ga, sem_wa)
                compute(g_a, w_a, m_a)
                pltpu.sync_copy(m_a, acc.at[idxi_blk.at[r0]], add=True)

                @pl.when(k < NBK // 2 - 1)
                def _prefetch():
                    issue(r0 + 2, eb0 + 2 * CH, g_a, w_a, sem_ga, sem_wa)

                wait(g_b, w_b, sem_gb, sem_wb)
                compute(g_b, w_b, m_b)
                pltpu.sync_copy(m_b, acc.at[idxi_blk.at[r0 + 1]], add=True)
                return icarry

            lax.fori_loop(0, NBK // 2, pair, 0)
            return carry

        lax.fori_loop(0, NBLK, block, 0)

    def copy_out(tid):
        @pl.when(sid != NS - 1)
        def _copy_full():
            pltpu.sync_copy(acc.at[pl.ds(nbase, NROW)],
                            out_hbm.at[tid, pl.ds(nbase, NROW)])

        @pl.when(sid == NS - 1)
        def _copy_tail():
            pltpu.sync_copy(acc.at[pl.ds(nbase, NROW_LAST)],
                            out_hbm.at[tid, pl.ds(nbase, NROW_LAST)])

    for q in range(NT // NC):
        tid = cid * (NT // NC) + q
        zero_acc()
        plsc.subcore_barrier()
        run_task(tid)
        plsc.subcore_barrier()
        copy_out(tid)


def kernel(s, v, radial_embeddings_1, radial_embeddings_2, f_cut_1, f_cut_2,
           unit_vectors_1, unit_vectors_2, edge_index, W1, b1, W2, b2, Wr, br):
    idx_i = edge_index[0].astype(jnp.int32)
    idx_j = edge_index[1].astype(jnp.int32)
    idxi2d = idx_i.reshape(NS, NBLK, NBK, CH)
    idxj4 = (idx_j.reshape(NROWS_ALL, CH)[None]
             + (jnp.arange(NT, dtype=jnp.int32) * N)[:, None, None]
             ).reshape(NT, NS, NBLK, NBK, CH)
    fc1 = f_cut_1.reshape(E, 1)
    fc2 = f_cut_2.reshape(E, 1)
    u1 = unit_vectors_1.reshape(E, 1)
    u2 = unit_vectors_2.reshape(E, 1)

    g4 = pl.pallas_call(
        _node_tc_body,
        grid=(N // BN,),
        in_specs=[
            pl.BlockSpec((BN, F), lambda i: (i, 0)),
            pl.BlockSpec((BN, F), lambda i: (i, 0)),
            pl.BlockSpec((F, F), lambda i: (0, 0)),
            pl.BlockSpec((1, F), lambda i: (0, 0)),
            pl.BlockSpec((F, R4F), lambda i: (0, 0)),
            pl.BlockSpec((1, R4F), lambda i: (0, 0)),
        ],
        out_specs=pl.BlockSpec((NT, BN, F), lambda i: (0, i, 0)),
        out_shape=jax.ShapeDtypeStruct((NT, N, F), jnp.float32),
    )(s, v, W1.T, b1.reshape(1, F), W2.T, b2.reshape(1, R4F))

    w4 = pl.pallas_call(
        _edge_tc_body,
        grid=(E // BE,),
        in_specs=[
            pl.BlockSpec((BE, R), lambda i: (i, 0)),
            pl.BlockSpec((BE, R), lambda i: (i, 0)),
            pl.BlockSpec((BE, 1), lambda i: (i, 0)),
            pl.BlockSpec((BE, 1), lambda i: (i, 0)),
            pl.BlockSpec((BE, 1), lambda i: (i, 0)),
            pl.BlockSpec((BE, 1), lambda i: (i, 0)),
            pl.BlockSpec((R, R4F), lambda i: (0, 0)),
            pl.BlockSpec((1, R4F), lambda i: (0, 0)),
        ],
        out_specs=pl.BlockSpec((NT, BE, F), lambda i: (0, i, 0)),
        out_shape=jax.ShapeDtypeStruct((NT, E, F), jnp.float32),
    )(radial_embeddings_1, radial_embeddings_2, fc1, fc2, u1, u2,
      Wr.T, br.reshape(1, R4F))

    p4 = _sc_scatter4(g4.reshape(NT * N, F), w4.reshape(NT * E, F),
                      idxj4, idxi2d)

    out_s, out_v = pl.pallas_call(
        _combine_tc_body,
        grid=(N // BN,),
        in_specs=[
            pl.BlockSpec((BN, F), lambda i: (i, 0)),
            pl.BlockSpec((BN, F), lambda i: (i, 0)),
            pl.BlockSpec((NT, BN, F), lambda i: (0, i, 0)),
        ],
        out_specs=[
            pl.BlockSpec((BN, F), lambda i: (i, 0)),
            pl.BlockSpec((BN, F), lambda i: (i, 0)),
        ],
        out_shape=[
            jax.ShapeDtypeStruct((N, F), jnp.float32),
            jax.ShapeDtypeStruct((N, F), jnp.float32),
        ],
    )(s, v, p4)

    return out_s, out_v
